# baseline (speedup 1.0000x reference)
import functools

import jax
import jax.numpy as jnp
from jax import lax
from jax.experimental import pallas as pl
from jax.experimental.pallas import tpu as pltpu
from jax.experimental.pallas import tpu_sc as plsc


@functools.cache
def _build(n_atoms, num_rows, dim):
    info = plsc.get_sparse_core_info()
    nc, ns, nl = info.num_cores, info.num_subcores, info.num_lanes
    nw = nc * ns
    per_w = -(-n_atoms // nw)
    per_w = -(-per_w // (2 * nl)) * (2 * nl)
    c_rows = per_w
    for cand in (160, 112, 96, 64, 32, 16):
        if per_w % cand == 0 and cand % nl == 0:
            c_rows = cand
            break
    nchunk = per_w // c_rows
    nbuf = 2
    assert nchunk % nbuf == 0
    nround = nchunk // nbuf
    groups = c_rows // nl
    last_base = n_atoms - per_w
    assert last_base % 8 == 0 and last_base >= 0
    ncg = dim // nl

    mesh = plsc.VectorSubcoreMesh(core_axis_name="c", subcore_axis_name="s")

    @functools.partial(
        pl.kernel,
        out_type=jax.ShapeDtypeStruct((n_atoms, dim), jnp.float32),
        mesh=mesh,
        compiler_params=pltpu.CompilerParams(needs_layout_passes=False),
        scratch_types=[
            pltpu.VMEM((num_rows + 1, dim), jnp.float32),
            pltpu.VMEM((per_w,), jnp.int32),
        ]
        + [pltpu.VMEM((c_rows, dim), jnp.float32) for _ in range(nbuf)]
        + [pltpu.SemaphoreType.DMA for _ in range(nbuf)],
    )
    def k(idx_hbm, table_hbm, out_hbm, table_t, idx_t, *bufsems):
        bufs = list(bufsems[:nbuf])
        ssems = list(bufsems[nbuf:])
        wid = lax.axis_index("s") * nc + lax.axis_index("c")
        base = jnp.minimum(wid * per_w, last_base)
        pltpu.sync_copy(table_hbm, table_t.at[pl.ds(1, num_rows)])
        pltpu.sync_copy(idx_hbm.at[pl.ds(base, per_w)], idx_t)

        zeros16 = jnp.zeros((nl,), jnp.int32)
        cols = [lax.iota(jnp.int32, nl) + cg * nl for cg in range(ncg)]

        def fill(buf, chunk):
            @pl.loop(0, groups)
            def _(g):
                pos = chunk * c_rows + g * nl
                idxv = idx_t[pl.ds(pos, nl)]
                for r in range(nl):
                    row_splat = idxv.at[zeros16 + r].get(
                        mode="promise_in_bounds"
                    )
                    lrow = g * nl + r
                    for cg in range(ncg):
                        vals = plsc.load_gather(table_t, [row_splat, cols[cg]])
                        plsc.store_scatter(
                            buf, [zeros16 + lrow, cols[cg]], vals
                        )

        def store(buf, sem, chunk):
            return pltpu.async_copy(
                buf, out_hbm.at[pl.ds(base + chunk * c_rows, c_rows)], sem
            )

        for b in range(nbuf):
            fill(bufs[b], b)
            store(bufs[b], ssems[b], b)

        @pl.loop(1, nround)
        def _(rd):
            for b in range(nbuf):
                chunk = rd * nbuf + b
                pltpu.make_async_copy(
                    bufs[b],
                    out_hbm.at[pl.ds(base + (chunk - nbuf) * c_rows, c_rows)],
                    ssems[b],
                ).wait()
                fill(bufs[b], chunk)
                store(bufs[b], ssems[b], chunk)

        for b in range(nbuf):
            chunk = (nround - 1) * nbuf + b
            pltpu.make_async_copy(
                bufs[b],
                out_hbm.at[pl.ds(base + chunk * c_rows, c_rows)],
                ssems[b],
            ).wait()

    def run(atomic_number, embeddings):
        return k(atomic_number, embeddings)

    return run


def kernel(atomic_number, embeddings):
    return _build(atomic_number.shape[0], embeddings.shape[0], embeddings.shape[1])(
        atomic_number, embeddings
    )


# speedup vs baseline: 1.3131x; 1.3131x over previous
import functools

import jax
import jax.numpy as jnp
from jax import lax
from jax.experimental import pallas as pl
from jax.experimental.pallas import tpu as pltpu
from jax.experimental.pallas import tpu_sc as plsc


@functools.cache
def _build(n_atoms, num_rows, dim):
    info = plsc.get_sparse_core_info()
    nc, ns, nl = info.num_cores, info.num_subcores, info.num_lanes
    nw = nc * ns
    per_w = -(-n_atoms // nw)
    per_w = -(-per_w // (2 * nl)) * (2 * nl)
    c_rows = per_w
    for cand in (160, 112, 96, 64, 32, 16):
        if per_w % cand == 0 and cand % nl == 0:
            c_rows = cand
            break
    nchunk = per_w // c_rows
    nbuf = 2
    assert nchunk % nbuf == 0
    nround = nchunk // nbuf
    groups = c_rows // nl
    last_base = n_atoms - per_w
    assert last_base % 8 == 0 and last_base >= 0
    ncg = dim // nl

    mesh = plsc.VectorSubcoreMesh(core_axis_name="c", subcore_axis_name="s")

    @functools.partial(
        pl.kernel,
        out_type=jax.ShapeDtypeStruct((n_atoms, dim), jnp.float32),
        mesh=mesh,
        compiler_params=pltpu.CompilerParams(needs_layout_passes=False),
        scratch_types=[
            pltpu.VMEM_SHARED((num_rows + 1, dim), jnp.float32),
            pltpu.VMEM((num_rows + 1, dim), jnp.float32),
            pltpu.VMEM((per_w,), jnp.int32),
        ]
        + [pltpu.VMEM((c_rows, dim), jnp.float32) for _ in range(nbuf)]
        + [pltpu.SemaphoreType.DMA for _ in range(nbuf)],
    )
    def k(idx_hbm, table_hbm, out_hbm, table_s, table_t, idx_t, *bufsems):
        bufs = list(bufsems[:nbuf])
        ssems = list(bufsems[nbuf:])
        sid = lax.axis_index("s")
        wid = sid * nc + lax.axis_index("c")
        base = jnp.minimum(wid * per_w, last_base)
        @pl.when(sid == 0)
        def _():
            pltpu.sync_copy(table_hbm, table_s.at[pl.ds(1, num_rows)])

        pltpu.sync_copy(idx_hbm.at[pl.ds(base, per_w)], idx_t)
        plsc.subcore_barrier()
        pltpu.sync_copy(table_s, table_t)

        zeros16 = jnp.zeros((nl,), jnp.int32)
        cols = [lax.iota(jnp.int32, nl) + cg * nl for cg in range(ncg)]

        def fill(buf, chunk):
            @plsc.parallel_loop(0, groups)
            def _(g):
                pos = chunk * c_rows + g * nl
                idxv = idx_t[pl.ds(pos, nl)]
                for r in range(nl):
                    row_splat = idxv.at[zeros16 + r].get(
                        mode="promise_in_bounds"
                    )
                    lrow = g * nl + r
                    for cg in range(ncg):
                        vals = plsc.load_gather(table_t, [row_splat, cols[cg]])
                        plsc.store_scatter(
                            buf, [zeros16 + lrow, cols[cg]], vals
                        )

        def store(buf, sem, chunk):
            return pltpu.async_copy(
                buf, out_hbm.at[pl.ds(base + chunk * c_rows, c_rows)], sem
            )

        for b in range(nbuf):
            fill(bufs[b], b)
            store(bufs[b], ssems[b], b)

        @pl.loop(1, nround)
        def _(rd):
            for b in range(nbuf):
                chunk = rd * nbuf + b
                pltpu.make_async_copy(
                    bufs[b],
                    out_hbm.at[pl.ds(base + (chunk - nbuf) * c_rows, c_rows)],
                    ssems[b],
                ).wait()
                fill(bufs[b], chunk)
                store(bufs[b], ssems[b], chunk)

        for b in range(nbuf):
            chunk = (nround - 1) * nbuf + b
            pltpu.make_async_copy(
                bufs[b],
                out_hbm.at[pl.ds(base + chunk * c_rows, c_rows)],
                ssems[b],
            ).wait()

    def run(atomic_number, embeddings):
        return k(atomic_number, embeddings)

    return run


def kernel(atomic_number, embeddings):
    return _build(atomic_number.shape[0], embeddings.shape[0], embeddings.shape[1])(
        atomic_number, embeddings
    )


# speedup vs baseline: 2.7365x; 2.0840x over previous
import functools

import jax
import jax.numpy as jnp
from jax import lax
from jax.experimental import pallas as pl
from jax.experimental.pallas import tpu as pltpu
from jax.experimental.pallas import tpu_sc as plsc


@functools.cache
def _build(n_atoms, num_rows, dim):
    info = plsc.get_sparse_core_info()
    nc, ns = info.num_cores, info.num_subcores
    nw = nc * ns
    per_w = -(-n_atoms // nw)
    per_w = -(-per_w // 8) * 8
    c_rows = 184
    nchunk = per_w // c_rows
    assert nchunk * c_rows == per_w
    last_base = n_atoms - per_w
    assert last_base % 8 == 0 and last_base >= 0

    mesh = plsc.VectorSubcoreMesh(core_axis_name="c", subcore_axis_name="s")

    nbuf = 4
    depth = nbuf - 2

    @functools.partial(
        pl.kernel,
        out_type=jax.ShapeDtypeStruct((n_atoms, dim), jnp.float32),
        mesh=mesh,
        scratch_types=[
            pltpu.VMEM_SHARED((num_rows + 1, dim), jnp.float32),
            pltpu.VMEM((per_w,), jnp.int32),
        ]
        + [pltpu.VMEM((c_rows, dim), jnp.float32) for _ in range(nbuf)]
        + [pltpu.SemaphoreType.DMA for _ in range(2 * nbuf)],
    )
    def k(idx_hbm, table_hbm, out_hbm, table_s, idx_v, *bufsems):
        bufs = list(bufsems[:nbuf])
        gsems = list(bufsems[nbuf : 2 * nbuf])
        ssems = list(bufsems[2 * nbuf :])
        sid = lax.axis_index("s")
        wid = sid * nc + lax.axis_index("c")
        base = jnp.minimum(wid * per_w, last_base)
        @pl.when(sid == 0)
        def _():
            pltpu.sync_copy(table_hbm, table_s.at[pl.ds(1, num_rows)])

        pltpu.sync_copy(idx_hbm.at[pl.ds(base, per_w)], idx_v)
        plsc.subcore_barrier()

        def gather(c, b):
            return pltpu.async_copy(
                table_s.at[idx_v.at[pl.ds(c * c_rows, c_rows)]], bufs[b], gsems[b]
            )

        def store(c, b):
            return pltpu.async_copy(
                bufs[b], out_hbm.at[pl.ds(base + c * c_rows, c_rows)], ssems[b]
            )

        ghandle = [None] * nbuf
        shandle = [None] * nbuf
        for b in range(depth):
            ghandle[b] = gather(b, b)
        for c in range(nchunk):
            cb = c % nbuf
            gn = c + depth
            if gn < nchunk:
                gb = gn % nbuf
                if shandle[gb] is not None:
                    shandle[gb].wait()
                ghandle[gb] = gather(gn, gb)
            ghandle[cb].wait()
            shandle[cb] = store(c, cb)
        for h in shandle:
            if h is not None:
                h.wait()

    def run(atomic_number, embeddings):
        return k(atomic_number, embeddings)

    return run


def kernel(atomic_number, embeddings):
    return _build(atomic_number.shape[0], embeddings.shape[0], embeddings.shape[1])(
        atomic_number, embeddings
    )
